# Initial kernel scaffold; baseline (speedup 1.0000x reference)
import jax
import jax.numpy as jnp
from jax.experimental import pallas as pl


def kernel(node_feats, edge_feats, senders, receivers, We, be, Wn, bn):
    raise NotImplementedError("write your pallas kernel here")


# speedup vs baseline: 2.9390x; 2.9390x over previous
import functools

import jax
import jax.numpy as jnp
from jax import lax
from jax.experimental import pallas as pl
from jax.experimental.pallas import tpu as pltpu
from jax.experimental.pallas import tpu_sc as plsc

NC = 2
NS = 16
NW = NC * NS


def _dense1_body(x_ref, ws_ref, wr_ref, ps_ref, pr_ref):
    x = x_ref[...]
    ps_ref[...] = jnp.dot(x, ws_ref[...], preferred_element_type=jnp.float32)
    pr_ref[...] = jnp.dot(x, wr_ref[...], preferred_element_type=jnp.float32)


def _dense1(node_feats, we_s, we_r):
    n, d = node_feats.shape
    blk = 1000
    grid = (n // blk,)
    return pl.pallas_call(
        _dense1_body,
        grid=grid,
        in_specs=[
            pl.BlockSpec((blk, d), lambda i: (i, 0)),
            pl.BlockSpec((d, d), lambda i: (0, 0)),
            pl.BlockSpec((d, d), lambda i: (0, 0)),
        ],
        out_specs=[
            pl.BlockSpec((blk, d), lambda i: (i, 0)),
            pl.BlockSpec((blk, d), lambda i: (i, 0)),
        ],
        out_shape=[
            jax.ShapeDtypeStruct((n, d), jnp.float32),
            jax.ShapeDtypeStruct((n, d), jnp.float32),
        ],
    )(node_feats, we_s, we_r)


def _make_gather(n_edges, n_nodes, d):
    per_w = n_edges // NW
    ch = 80
    n_ch = per_w // ch
    mesh = plsc.VectorSubcoreMesh(
        core_axis_name="c", subcore_axis_name="s", num_cores=NC, num_subcores=NS)

    @functools.partial(
        pl.kernel,
        mesh=mesh,
        out_type=(
            jax.ShapeDtypeStruct((n_edges, d), jnp.float32),
            jax.ShapeDtypeStruct((n_edges, d), jnp.float32),
        ),
        scratch_types=[
            pltpu.VMEM((ch,), jnp.int32),
            pltpu.VMEM((ch,), jnp.int32),
            pltpu.VMEM((ch, d), jnp.float32),
            pltpu.VMEM((ch, d), jnp.float32),
            pltpu.SemaphoreType.DMA,
            pltpu.SemaphoreType.DMA,
        ],
    )
    def gather(ps_hbm, pr_hbm, s_hbm, r_hbm, gs_hbm, gr_hbm,
               idx_s, idx_r, rows_s, rows_r, sem_s, sem_r):
        wid = lax.axis_index("s") * NC + lax.axis_index("c")
        base = wid * per_w

        def body(i, carry):
            off = base + i * ch
            pltpu.sync_copy(s_hbm.at[pl.ds(off, ch)], idx_s)
            pltpu.sync_copy(r_hbm.at[pl.ds(off, ch)], idx_r)
            c1 = pltpu.async_copy(ps_hbm.at[idx_s], rows_s, sem_s)
            c2 = pltpu.async_copy(pr_hbm.at[idx_r], rows_r, sem_r)
            c1.wait()
            c2.wait()
            pltpu.sync_copy(rows_s, gs_hbm.at[pl.ds(off, ch)])
            pltpu.sync_copy(rows_r, gr_hbm.at[pl.ds(off, ch)])
            return carry

        lax.fori_loop(0, n_ch, body, 0)

    return gather


def _edge_body(e_ref, we_ref, b_ref, gs_ref, gr_ref, o_ref):
    o_ref[...] = (
        jnp.dot(e_ref[...], we_ref[...], preferred_element_type=jnp.float32)
        + b_ref[...] + gs_ref[...] + gr_ref[...])


def _edge_out(edge_feats, we_e, be, g_s, g_r):
    m, de = edge_feats.shape
    d = we_e.shape[1]
    blk = 4000
    grid = (m // blk,)
    return pl.pallas_call(
        _edge_body,
        grid=grid,
        in_specs=[
            pl.BlockSpec((blk, de), lambda i: (i, 0)),
            pl.BlockSpec((de, d), lambda i: (0, 0)),
            pl.BlockSpec((1, d), lambda i: (0, 0)),
            pl.BlockSpec((blk, d), lambda i: (i, 0)),
            pl.BlockSpec((blk, d), lambda i: (i, 0)),
        ],
        out_specs=pl.BlockSpec((blk, d), lambda i: (i, 0)),
        out_shape=jax.ShapeDtypeStruct((m, d), jnp.float32),
    )(edge_feats, we_e, be, g_s, g_r)


def _make_scatter(n_edges, n_nodes, d):
    half = n_edges // NC
    per_t = half // NS
    ch = 80
    n_ch = per_t // ch
    rows_t = -(n_nodes // -NS)
    rows_t += (-rows_t) % ch
    rows_last = n_nodes - (NS - 1) * rows_t
    assert rows_last > 0 and rows_last % 8 == 0
    assert rows_t % ch == 0 and rows_last % ch == 0
    n_init, n_init_last = rows_t // ch, rows_last // ch
    mesh = plsc.VectorSubcoreMesh(
        core_axis_name="c", subcore_axis_name="s", num_cores=NC, num_subcores=NS)

    @functools.partial(
        pl.kernel,
        mesh=mesh,
        out_type=jax.ShapeDtypeStruct((NC, n_nodes, d), jnp.float32),
        scratch_types=[
            pltpu.VMEM((ch,), jnp.int32),
            pltpu.VMEM((ch, d), jnp.float32),
            pltpu.VMEM_SHARED((n_nodes, d), jnp.float32),
            pltpu.SemaphoreType.DMA,
        ],
    )
    def scatter(ne_hbm, idx_hbm, z_hbm, agg_hbm, idx_v, buf_v, acc_sh, sem):
        cid = lax.axis_index("c")
        sid = lax.axis_index("s")
        row0 = sid * rows_t
        n_my = lax.select(sid == NS - 1, n_init_last, n_init)

        def init_chunk(j, carry):
            r0 = row0 + j * ch
            pltpu.sync_copy(z_hbm.at[pl.ds(r0, ch)], buf_v)
            pltpu.sync_copy(buf_v, acc_sh.at[pl.ds(r0, ch)])
            return carry

        lax.fori_loop(0, n_my, init_chunk, 0)
        plsc.subcore_barrier()

        base = cid * half + sid * per_t

        def body(i, carry):
            off = base + i * ch
            pltpu.sync_copy(idx_hbm.at[pl.ds(off, ch)], idx_v)
            pltpu.sync_copy(ne_hbm.at[pl.ds(off, ch)], buf_v)
            pltpu.sync_copy(buf_v, acc_sh.at[idx_v], add=True)
            return carry

        lax.fori_loop(0, n_ch, body, 0)
        plsc.subcore_barrier()

        def wb_chunk(j, carry):
            r0 = row0 + j * ch
            pltpu.sync_copy(acc_sh.at[pl.ds(r0, ch)], buf_v)
            pltpu.sync_copy(buf_v, agg_hbm.at[cid, pl.ds(r0, ch)])
            return carry

        lax.fori_loop(0, n_my, wb_chunk, 0)

    return scatter


def _dense2_body(x_ref, s_ref, r_ref, wn_ref, ws_ref, wr_ref, b_ref, o_ref):
    sent = s_ref[0] + s_ref[1]
    recv = r_ref[0] + r_ref[1]
    o_ref[...] = (
        jnp.dot(x_ref[...], wn_ref[...], preferred_element_type=jnp.float32)
        + jnp.dot(sent, ws_ref[...], preferred_element_type=jnp.float32)
        + jnp.dot(recv, wr_ref[...], preferred_element_type=jnp.float32)
        + b_ref[...])


def _dense2(node_feats, agg_s, agg_r, wn_n, wn_s, wn_r, bn):
    n, d = node_feats.shape
    blk = 1000
    grid = (n // blk,)
    row = pl.BlockSpec((blk, d), lambda i: (i, 0))
    prow = pl.BlockSpec((NC, blk, d), lambda i: (0, i, 0))
    w = pl.BlockSpec((d, d), lambda i: (0, 0))
    return pl.pallas_call(
        _dense2_body,
        grid=grid,
        in_specs=[row, prow, prow, w, w, w, pl.BlockSpec((1, d), lambda i: (0, 0))],
        out_specs=pl.BlockSpec((blk, d), lambda i: (i, 0)),
        out_shape=jax.ShapeDtypeStruct((n, d), jnp.float32),
    )(node_feats, agg_s, agg_r, wn_n, wn_s, wn_r, bn)


def kernel(node_feats, edge_feats, senders, receivers, We, be, Wn, bn):
    n_nodes, d = node_feats.shape
    n_edges, d_edge = edge_feats.shape

    we_e = We[:d_edge]
    we_s = We[d_edge:d_edge + d]
    we_r = We[d_edge + d:]
    wn_n = Wn[:d]
    wn_s = Wn[d:2 * d]
    wn_r = Wn[2 * d:]

    p_s, p_r = _dense1(node_feats, we_s, we_r)
    g_s, g_r = _make_gather(n_edges, n_nodes, d)(p_s, p_r, senders, receivers)
    new_edge = _edge_out(edge_feats, we_e, be.reshape(1, d), g_s, g_r)
    zeros = jnp.zeros((n_nodes, d), jnp.float32)
    scat = _make_scatter(n_edges, n_nodes, d)
    agg_s = scat(new_edge, senders, zeros)
    agg_r = scat(new_edge, receivers, zeros)
    new_node = _dense2(node_feats, agg_s, agg_r, wn_n, wn_s, wn_r,
                       bn.reshape(1, d))
    return (new_edge, new_node)


# speedup vs baseline: 3.6388x; 1.2381x over previous
import functools

import jax
import jax.numpy as jnp
from jax import lax
from jax.experimental import pallas as pl
from jax.experimental.pallas import tpu as pltpu
from jax.experimental.pallas import tpu_sc as plsc

NC = 2
NS = 16
NW = NC * NS


def _dense1_body(x_ref, ws_ref, wr_ref, ps_ref, pr_ref):
    x = x_ref[...]
    ps_ref[...] = jnp.dot(x, ws_ref[...], preferred_element_type=jnp.float32)
    pr_ref[...] = jnp.dot(x, wr_ref[...], preferred_element_type=jnp.float32)


def _dense1(node_feats, we_s, we_r):
    n, d = node_feats.shape
    blk = 1000
    grid = (n // blk,)
    return pl.pallas_call(
        _dense1_body,
        grid=grid,
        in_specs=[
            pl.BlockSpec((blk, d), lambda i: (i, 0)),
            pl.BlockSpec((d, d), lambda i: (0, 0)),
            pl.BlockSpec((d, d), lambda i: (0, 0)),
        ],
        out_specs=[
            pl.BlockSpec((blk, d), lambda i: (i, 0)),
            pl.BlockSpec((blk, d), lambda i: (i, 0)),
        ],
        out_shape=[
            jax.ShapeDtypeStruct((n, d), jnp.float32),
            jax.ShapeDtypeStruct((n, d), jnp.float32),
        ],
    )(node_feats, we_s, we_r)


def _make_gather(n_edges, n_nodes, d):
    per_w = n_edges // NW
    ch = 40
    n_ch = per_w // ch
    assert n_ch % 2 == 0
    n_grp = n_ch // 2
    mesh = plsc.VectorSubcoreMesh(
        core_axis_name="c", subcore_axis_name="s", num_cores=NC, num_subcores=NS)

    @functools.partial(
        pl.kernel,
        mesh=mesh,
        out_type=jax.ShapeDtypeStruct((n_edges, d), jnp.float32),
        scratch_types=[
            pltpu.VMEM((per_w,), jnp.int32),
            pltpu.VMEM((per_w,), jnp.int32),
            pltpu.VMEM((ch, d), jnp.float32),
            pltpu.VMEM((ch, d), jnp.float32),
            pltpu.VMEM((ch, d), jnp.float32),
            pltpu.VMEM((ch, d), jnp.float32),
            pltpu.VMEM((ch, d), jnp.float32),
            pltpu.VMEM((ch, d), jnp.float32),
            pltpu.SemaphoreType.DMA,
            pltpu.SemaphoreType.DMA,
            pltpu.SemaphoreType.DMA,
            pltpu.SemaphoreType.DMA,
            pltpu.SemaphoreType.DMA,
            pltpu.SemaphoreType.DMA,
        ],
    )
    def gather(ps_hbm, pr_hbm, s_hbm, r_hbm, g_hbm,
               idx_s, idx_r, rs0, rs1, rr0, rr1, go0, go1,
               sem_s0, sem_s1, sem_r0, sem_r1, sem_w0, sem_w1):
        wid = lax.axis_index("s") * NC + lax.axis_index("c")
        base = wid * per_w

        pltpu.sync_copy(s_hbm.at[pl.ds(base, per_w)], idx_s)
        pltpu.sync_copy(r_hbm.at[pl.ds(base, per_w)], idx_r)

        def fire(c, rs, rr, sem_a, sem_b):
            off = c * ch
            pltpu.async_copy(ps_hbm.at[idx_s.at[pl.ds(off, ch)]], rs, sem_a)
            pltpu.async_copy(pr_hbm.at[idx_r.at[pl.ds(off, ch)]], rr, sem_b)

        def drain(rs, rr, sem_a, sem_b):
            pltpu.make_async_copy(ps_hbm.at[idx_s.at[pl.ds(0, ch)]], rs, sem_a).wait()
            pltpu.make_async_copy(pr_hbm.at[idx_r.at[pl.ds(0, ch)]], rr, sem_b).wait()

        def add(rs, rr, go):
            def row(j, carry):
                for k in range(d // 16):
                    sl = pl.ds(k * 16, 16)
                    go[j, sl] = rs[j, sl] + rr[j, sl]
                return carry

            lax.fori_loop(0, ch, row, 0)

        def fire_write(c, go, sem_w):
            pltpu.async_copy(go, g_hbm.at[pl.ds(base + c * ch, ch)], sem_w)

        def wait_write(go, sem_w):
            pltpu.make_async_copy(go, g_hbm.at[pl.ds(0, ch)], sem_w).wait()

        fire(0, rs0, rr0, sem_s0, sem_r0)

        def group(p, carry):
            c0 = 2 * p
            fire(c0 + 1, rs1, rr1, sem_s1, sem_r1)
            drain(rs0, rr0, sem_s0, sem_r0)

            @pl.when(p > 0)
            def _():
                wait_write(go0, sem_w0)

            add(rs0, rr0, go0)
            fire_write(c0, go0, sem_w0)

            @pl.when(p < n_grp - 1)
            def _():
                fire(c0 + 2, rs0, rr0, sem_s0, sem_r0)

            drain(rs1, rr1, sem_s1, sem_r1)

            @pl.when(p > 0)
            def _():
                wait_write(go1, sem_w1)

            add(rs1, rr1, go1)
            fire_write(c0 + 1, go1, sem_w1)
            return carry

        lax.fori_loop(0, n_grp, group, 0)
        wait_write(go0, sem_w0)
        wait_write(go1, sem_w1)

    return gather


def _edge_body(e_ref, we_ref, b_ref, g_ref, o_ref):
    o_ref[...] = (
        jnp.dot(e_ref[...], we_ref[...], preferred_element_type=jnp.float32)
        + b_ref[...] + g_ref[...])


def _edge_out(edge_feats, we_e, be, g):
    m, de = edge_feats.shape
    d = we_e.shape[1]
    blk = 4000
    grid = (m // blk,)
    return pl.pallas_call(
        _edge_body,
        grid=grid,
        in_specs=[
            pl.BlockSpec((blk, de), lambda i: (i, 0)),
            pl.BlockSpec((de, d), lambda i: (0, 0)),
            pl.BlockSpec((1, d), lambda i: (0, 0)),
            pl.BlockSpec((blk, d), lambda i: (i, 0)),
        ],
        out_specs=pl.BlockSpec((blk, d), lambda i: (i, 0)),
        out_shape=jax.ShapeDtypeStruct((m, d), jnp.float32),
    )(edge_feats, we_e, be, g)


def _make_scatter(n_edges, n_nodes, d):
    half = n_edges // NC
    per_t = half // NS
    ch = 80
    n_ch = per_t // ch
    rows_t = -(n_nodes // -NS)
    rows_t += (-rows_t) % ch
    rows_last = n_nodes - (NS - 1) * rows_t
    assert rows_last > 0 and rows_last % 8 == 0
    assert rows_t % ch == 0 and rows_last % ch == 0
    n_init, n_init_last = rows_t // ch, rows_last // ch
    mesh = plsc.VectorSubcoreMesh(
        core_axis_name="c", subcore_axis_name="s", num_cores=NC, num_subcores=NS)

    @functools.partial(
        pl.kernel,
        mesh=mesh,
        out_type=jax.ShapeDtypeStruct((NC, n_nodes, d), jnp.float32),
        scratch_types=[
            pltpu.VMEM((ch,), jnp.int32),
            pltpu.VMEM((ch, d), jnp.float32),
            pltpu.VMEM_SHARED((n_nodes, d), jnp.float32),
            pltpu.SemaphoreType.DMA,
        ],
    )
    def scatter(ne_hbm, idx_hbm, z_hbm, agg_hbm, idx_v, buf_v, acc_sh, sem):
        cid = lax.axis_index("c")
        sid = lax.axis_index("s")
        row0 = sid * rows_t
        n_my = lax.select(sid == NS - 1, n_init_last, n_init)

        def init_chunk(j, carry):
            r0 = row0 + j * ch
            pltpu.sync_copy(z_hbm.at[pl.ds(r0, ch)], buf_v)
            pltpu.sync_copy(buf_v, acc_sh.at[pl.ds(r0, ch)])
            return carry

        lax.fori_loop(0, n_my, init_chunk, 0)
        plsc.subcore_barrier()

        base = cid * half + sid * per_t

        def body(i, carry):
            off = base + i * ch
            pltpu.sync_copy(idx_hbm.at[pl.ds(off, ch)], idx_v)
            pltpu.sync_copy(ne_hbm.at[pl.ds(off, ch)], buf_v)
            pltpu.sync_copy(buf_v, acc_sh.at[idx_v], add=True)
            return carry

        lax.fori_loop(0, n_ch, body, 0)
        plsc.subcore_barrier()

        def wb_chunk(j, carry):
            r0 = row0 + j * ch
            pltpu.sync_copy(acc_sh.at[pl.ds(r0, ch)], buf_v)
            pltpu.sync_copy(buf_v, agg_hbm.at[cid, pl.ds(r0, ch)])
            return carry

        lax.fori_loop(0, n_my, wb_chunk, 0)

    return scatter


def _dense2_body(x_ref, s_ref, r_ref, wn_ref, ws_ref, wr_ref, b_ref, o_ref):
    sent = s_ref[0] + s_ref[1]
    recv = r_ref[0] + r_ref[1]
    o_ref[...] = (
        jnp.dot(x_ref[...], wn_ref[...], preferred_element_type=jnp.float32)
        + jnp.dot(sent, ws_ref[...], preferred_element_type=jnp.float32)
        + jnp.dot(recv, wr_ref[...], preferred_element_type=jnp.float32)
        + b_ref[...])


def _dense2(node_feats, agg_s, agg_r, wn_n, wn_s, wn_r, bn):
    n, d = node_feats.shape
    blk = 1000
    grid = (n // blk,)
    row = pl.BlockSpec((blk, d), lambda i: (i, 0))
    prow = pl.BlockSpec((NC, blk, d), lambda i: (0, i, 0))
    w = pl.BlockSpec((d, d), lambda i: (0, 0))
    return pl.pallas_call(
        _dense2_body,
        grid=grid,
        in_specs=[row, prow, prow, w, w, w, pl.BlockSpec((1, d), lambda i: (0, 0))],
        out_specs=pl.BlockSpec((blk, d), lambda i: (i, 0)),
        out_shape=jax.ShapeDtypeStruct((n, d), jnp.float32),
    )(node_feats, agg_s, agg_r, wn_n, wn_s, wn_r, bn)


def kernel(node_feats, edge_feats, senders, receivers, We, be, Wn, bn):
    n_nodes, d = node_feats.shape
    n_edges, d_edge = edge_feats.shape

    we_e = We[:d_edge]
    we_s = We[d_edge:d_edge + d]
    we_r = We[d_edge + d:]
    wn_n = Wn[:d]
    wn_s = Wn[d:2 * d]
    wn_r = Wn[2 * d:]

    p_s, p_r = _dense1(node_feats, we_s, we_r)
    g = _make_gather(n_edges, n_nodes, d)(p_s, p_r, senders, receivers)
    new_edge = _edge_out(edge_feats, we_e, be.reshape(1, d), g)
    zeros = jnp.zeros((n_nodes, d), jnp.float32)
    scat = _make_scatter(n_edges, n_nodes, d)
    agg_s = scat(new_edge, senders, zeros)
    agg_r = scat(new_edge, receivers, zeros)
    new_node = _dense2(node_feats, agg_s, agg_r, wn_n, wn_s, wn_r,
                       bn.reshape(1, d))
    return (new_edge, new_node)


# speedup vs baseline: 4.7499x; 1.3053x over previous
import functools

import jax
import jax.numpy as jnp
from jax import lax
from jax.experimental import pallas as pl
from jax.experimental.pallas import tpu as pltpu
from jax.experimental.pallas import tpu_sc as plsc

NC = 2
NS = 16
NW = NC * NS


def _dense1_body(x_ref, ws_ref, wr_ref, ps_ref, pr_ref):
    x = x_ref[...]
    ps_ref[...] = jnp.dot(x, ws_ref[...], preferred_element_type=jnp.float32)
    pr_ref[...] = jnp.dot(x, wr_ref[...], preferred_element_type=jnp.float32)


def _dense1(node_feats, we_s, we_r):
    n, d = node_feats.shape
    blk = 1000
    grid = (n // blk,)
    return pl.pallas_call(
        _dense1_body,
        grid=grid,
        in_specs=[
            pl.BlockSpec((blk, d), lambda i: (i, 0)),
            pl.BlockSpec((d, d), lambda i: (0, 0)),
            pl.BlockSpec((d, d), lambda i: (0, 0)),
        ],
        out_specs=[
            pl.BlockSpec((blk, d), lambda i: (i, 0)),
            pl.BlockSpec((blk, d), lambda i: (i, 0)),
        ],
        out_shape=[
            jax.ShapeDtypeStruct((n, d), jnp.float32),
            jax.ShapeDtypeStruct((n, d), jnp.float32),
        ],
    )(node_feats, we_s, we_r)


def _make_gather(n_edges, n_nodes, d):
    per_w = n_edges // NW
    ch = 40
    n_ch = per_w // ch
    assert n_ch % 2 == 0
    n_grp = n_ch // 2
    mesh = plsc.VectorSubcoreMesh(
        core_axis_name="c", subcore_axis_name="s", num_cores=NC, num_subcores=NS)

    @functools.partial(
        pl.kernel,
        mesh=mesh,
        out_type=jax.ShapeDtypeStruct((n_edges, d), jnp.float32),
        scratch_types=[
            pltpu.VMEM((per_w,), jnp.int32),
            pltpu.VMEM((per_w,), jnp.int32),
            pltpu.VMEM((ch, d), jnp.float32),
            pltpu.VMEM((ch, d), jnp.float32),
            pltpu.VMEM((ch, d), jnp.float32),
            pltpu.VMEM((ch, d), jnp.float32),
            pltpu.VMEM((ch, d), jnp.float32),
            pltpu.VMEM((ch, d), jnp.float32),
            pltpu.SemaphoreType.DMA,
            pltpu.SemaphoreType.DMA,
            pltpu.SemaphoreType.DMA,
            pltpu.SemaphoreType.DMA,
            pltpu.SemaphoreType.DMA,
            pltpu.SemaphoreType.DMA,
        ],
    )
    def gather(ps_hbm, pr_hbm, s_hbm, r_hbm, g_hbm,
               idx_s, idx_r, rs0, rs1, rr0, rr1, go0, go1,
               sem_s0, sem_s1, sem_r0, sem_r1, sem_w0, sem_w1):
        wid = lax.axis_index("s") * NC + lax.axis_index("c")
        base = wid * per_w

        pltpu.sync_copy(s_hbm.at[pl.ds(base, per_w)], idx_s)
        pltpu.sync_copy(r_hbm.at[pl.ds(base, per_w)], idx_r)

        def fire(c, rs, rr, sem_a, sem_b):
            off = c * ch
            pltpu.async_copy(ps_hbm.at[idx_s.at[pl.ds(off, ch)]], rs, sem_a)
            pltpu.async_copy(pr_hbm.at[idx_r.at[pl.ds(off, ch)]], rr, sem_b)

        def drain(rs, rr, sem_a, sem_b):
            pltpu.make_async_copy(ps_hbm.at[idx_s.at[pl.ds(0, ch)]], rs, sem_a).wait()
            pltpu.make_async_copy(pr_hbm.at[idx_r.at[pl.ds(0, ch)]], rr, sem_b).wait()

        def add(rs, rr, go):
            def row(j, carry):
                for k in range(d // 16):
                    sl = pl.ds(k * 16, 16)
                    go[j, sl] = rs[j, sl] + rr[j, sl]
                return carry

            lax.fori_loop(0, ch, row, 0)

        def fire_write(c, go, sem_w):
            pltpu.async_copy(go, g_hbm.at[pl.ds(base + c * ch, ch)], sem_w)

        def wait_write(go, sem_w):
            pltpu.make_async_copy(go, g_hbm.at[pl.ds(0, ch)], sem_w).wait()

        fire(0, rs0, rr0, sem_s0, sem_r0)

        def group(p, carry):
            c0 = 2 * p
            fire(c0 + 1, rs1, rr1, sem_s1, sem_r1)
            drain(rs0, rr0, sem_s0, sem_r0)

            @pl.when(p > 0)
            def _():
                wait_write(go0, sem_w0)

            add(rs0, rr0, go0)
            fire_write(c0, go0, sem_w0)

            @pl.when(p < n_grp - 1)
            def _():
                fire(c0 + 2, rs0, rr0, sem_s0, sem_r0)

            drain(rs1, rr1, sem_s1, sem_r1)

            @pl.when(p > 0)
            def _():
                wait_write(go1, sem_w1)

            add(rs1, rr1, go1)
            fire_write(c0 + 1, go1, sem_w1)
            return carry

        lax.fori_loop(0, n_grp, group, 0)
        wait_write(go0, sem_w0)
        wait_write(go1, sem_w1)

    return gather


def _edge_body(e_ref, we_ref, b_ref, g_ref, o_ref):
    o_ref[...] = (
        jnp.dot(e_ref[...], we_ref[...], preferred_element_type=jnp.float32)
        + b_ref[...] + g_ref[...])


def _edge_out(edge_feats, we_e, be, g):
    m, de = edge_feats.shape
    d = we_e.shape[1]
    blk = 4000
    grid = (m // blk,)
    return pl.pallas_call(
        _edge_body,
        grid=grid,
        in_specs=[
            pl.BlockSpec((blk, de), lambda i: (i, 0)),
            pl.BlockSpec((de, d), lambda i: (0, 0)),
            pl.BlockSpec((1, d), lambda i: (0, 0)),
            pl.BlockSpec((blk, d), lambda i: (i, 0)),
        ],
        out_specs=pl.BlockSpec((blk, d), lambda i: (i, 0)),
        out_shape=jax.ShapeDtypeStruct((m, d), jnp.float32),
    )(edge_feats, we_e, be, g)


def _make_scatter(n_edges, n_nodes, d):
    per_t = n_edges // NS
    ch = 80
    n_ch = per_t // ch
    assert n_ch % 2 == 0
    n_grp = n_ch // 2
    rows_t = -(n_nodes // -NS)
    rows_t += (-rows_t) % ch
    rows_last = n_nodes - (NS - 1) * rows_t
    assert rows_last > 0 and rows_last % 8 == 0
    assert rows_t % ch == 0 and rows_last % ch == 0
    n_init, n_init_last = rows_t // ch, rows_last // ch
    mesh = plsc.VectorSubcoreMesh(
        core_axis_name="c", subcore_axis_name="s", num_cores=NC, num_subcores=NS)

    @functools.partial(
        pl.kernel,
        mesh=mesh,
        out_type=jax.ShapeDtypeStruct((NC, n_nodes, d), jnp.float32),
        scratch_types=[
            pltpu.VMEM((ch,), jnp.int32),
            pltpu.VMEM((ch,), jnp.int32),
            pltpu.VMEM((ch, d), jnp.float32),
            pltpu.VMEM((ch, d), jnp.float32),
            pltpu.VMEM_SHARED((n_nodes, d), jnp.float32),
            pltpu.SemaphoreType.DMA,
            pltpu.SemaphoreType.DMA,
            pltpu.SemaphoreType.DMA,
            pltpu.SemaphoreType.DMA,
        ],
    )
    def scatter(ne_hbm, idx_hbm, z_hbm, agg_hbm,
                idx0, idx1, buf0, buf1, acc_sh,
                sem_l0, sem_l1, sem_c0, sem_c1):
        cid = lax.axis_index("c")
        sid = lax.axis_index("s")
        row0 = sid * rows_t
        n_my = lax.select(sid == NS - 1, n_init_last, n_init)

        def init_chunk(j, carry):
            r0 = row0 + j * ch
            pltpu.sync_copy(z_hbm.at[pl.ds(r0, ch)], buf0)
            pltpu.sync_copy(buf0, acc_sh.at[pl.ds(r0, ch)])
            return carry

        lax.fori_loop(0, n_my, init_chunk, 0)
        plsc.subcore_barrier()

        ebase = sid * per_t
        ibase = cid * n_edges + ebase

        def fire_loads(c, idx_v, buf_v, sem_l):
            off = c * ch
            pltpu.async_copy(idx_hbm.at[pl.ds(ibase + off, ch)], idx_v, sem_l)
            pltpu.async_copy(ne_hbm.at[pl.ds(ebase + off, ch)], buf_v, sem_l)

        def wait_loads(idx_v, buf_v, sem_l):
            pltpu.make_async_copy(idx_hbm.at[pl.ds(0, ch)], idx_v, sem_l).wait()
            pltpu.make_async_copy(ne_hbm.at[pl.ds(0, ch)], buf_v, sem_l).wait()

        def fire_scatter(idx_v, buf_v, sem_c):
            pltpu.async_copy(buf_v, acc_sh.at[idx_v], sem_c, add=True)

        def wait_scatter(buf_v, sem_c):
            pltpu.make_async_copy(ne_hbm.at[pl.ds(0, ch)], buf_v, sem_c).wait()

        fire_loads(0, idx0, buf0, sem_l0)

        def group(p, carry):
            c0 = 2 * p

            @pl.when(p > 0)
            def _():
                wait_scatter(buf1, sem_c1)

            fire_loads(c0 + 1, idx1, buf1, sem_l1)
            wait_loads(idx0, buf0, sem_l0)
            fire_scatter(idx0, buf0, sem_c0)
            wait_loads(idx1, buf1, sem_l1)
            fire_scatter(idx1, buf1, sem_c1)
            wait_scatter(buf0, sem_c0)

            @pl.when(p < n_grp - 1)
            def _():
                fire_loads(c0 + 2, idx0, buf0, sem_l0)

            return carry

        lax.fori_loop(0, n_grp, group, 0)
        wait_scatter(buf1, sem_c1)
        plsc.subcore_barrier()

        def wb_chunk(j, carry):
            r0 = row0 + j * ch
            pltpu.sync_copy(acc_sh.at[pl.ds(r0, ch)], buf0)
            pltpu.sync_copy(buf0, agg_hbm.at[cid, pl.ds(r0, ch)])
            return carry

        lax.fori_loop(0, n_my, wb_chunk, 0)

    return scatter


def _dense2_body(x_ref, a_ref, wn_ref, ws_ref, wr_ref, b_ref, o_ref):
    o_ref[...] = (
        jnp.dot(x_ref[...], wn_ref[...], preferred_element_type=jnp.float32)
        + jnp.dot(a_ref[0], ws_ref[...], preferred_element_type=jnp.float32)
        + jnp.dot(a_ref[1], wr_ref[...], preferred_element_type=jnp.float32)
        + b_ref[...])


def _dense2(node_feats, agg, wn_n, wn_s, wn_r, bn):
    n, d = node_feats.shape
    blk = 1000
    grid = (n // blk,)
    row = pl.BlockSpec((blk, d), lambda i: (i, 0))
    prow = pl.BlockSpec((NC, blk, d), lambda i: (0, i, 0))
    w = pl.BlockSpec((d, d), lambda i: (0, 0))
    return pl.pallas_call(
        _dense2_body,
        grid=grid,
        in_specs=[row, prow, w, w, w, pl.BlockSpec((1, d), lambda i: (0, 0))],
        out_specs=pl.BlockSpec((blk, d), lambda i: (i, 0)),
        out_shape=jax.ShapeDtypeStruct((n, d), jnp.float32),
    )(node_feats, agg, wn_n, wn_s, wn_r, bn)


def kernel(node_feats, edge_feats, senders, receivers, We, be, Wn, bn):
    n_nodes, d = node_feats.shape
    n_edges, d_edge = edge_feats.shape

    we_e = We[:d_edge]
    we_s = We[d_edge:d_edge + d]
    we_r = We[d_edge + d:]
    wn_n = Wn[:d]
    wn_s = Wn[d:2 * d]
    wn_r = Wn[2 * d:]

    p_s, p_r = _dense1(node_feats, we_s, we_r)
    g = _make_gather(n_edges, n_nodes, d)(p_s, p_r, senders, receivers)
    new_edge = _edge_out(edge_feats, we_e, be.reshape(1, d), g)
    zeros = jnp.zeros((n_nodes, d), jnp.float32)
    idx_cat = jnp.concatenate([senders, receivers])
    agg = _make_scatter(n_edges, n_nodes, d)(new_edge, idx_cat, zeros)
    new_node = _dense2(node_feats, agg, wn_n, wn_s, wn_r, bn.reshape(1, d))
    return (new_edge, new_node)
